# baseline (speedup 1.0000x reference)
import functools

import jax
import jax.numpy as jnp
from jax import lax
from jax.experimental import pallas as pl
from jax.experimental.pallas import tpu as pltpu
from jax.experimental.pallas import tpu_sc as plsc

_B = 4096
_D = 32
_R = 64
_NC = 2
_NS = 16
_NW = _NC * _NS
_BPW = _B // _NW

_BB = 512
_G = _B // _BB


def _sc_gather(users, pos_items, neg_items, heads, pos_tails, neg_tails,
               ue4, ie4, ke4):
    mesh = plsc.VectorSubcoreMesh(core_axis_name="c", subcore_axis_name="s")
    out_t = [jax.ShapeDtypeStruct((_B, _D), jnp.float32)] * 8

    @functools.partial(
        pl.kernel,
        mesh=mesh,
        out_type=out_t,
        compiler_params=pltpu.CompilerParams(needs_layout_passes=False),
        scratch_types=[
            pltpu.VMEM((_BPW,), jnp.int32),
            pltpu.VMEM((_BPW,), jnp.int32),
            pltpu.VMEM((_BPW,), jnp.int32),
            pltpu.VMEM((_BPW, 128), jnp.float32),
            pltpu.VMEM((_BPW, _D), jnp.float32),
            pltpu.SemaphoreType.DMA,
        ],
    )
    def k(users_h, pos_h, neg_h, heads_h, pt_h, nt_h,
          ue_h, ie_h, ke_h,
          u_o, pi_o, pkg_o, ni_o, nkg_o, h_o, pt_o, nt_o,
          idx_v, tidx_v, sidx_v, rows_v, out_v, sem):
        wid = lax.axis_index("s") * _NC + lax.axis_index("c")
        base = wid * _BPW
        jobs = (
            (users_h, ((ue_h, u_o),)),
            (pos_h, ((ie_h, pi_o), (ke_h, pkg_o))),
            (neg_h, ((ie_h, ni_o), (ke_h, nkg_o))),
            (heads_h, ((ke_h, h_o),)),
            (pt_h, ((ke_h, pt_o),)),
            (nt_h, ((ke_h, nt_o),)),
        )
        for idx_h, tabouts in jobs:
            pltpu.sync_copy(idx_h.at[pl.ds(base, _BPW)], idx_v)
            for kk in range(_BPW // 16):
                sl16 = pl.ds(kk * 16, 16)
                raw = idx_v[sl16]
                tidx_v[sl16] = lax.shift_right_logical(raw, 2)
                sidx_v[sl16] = (raw & 3) * _D
            for tab_h, out_h in tabouts:
                pltpu.async_copy(tab_h.at[tidx_v], rows_v, sem).wait()

                def body(i, _):
                    for g in range(_BPW // 16):
                        rows16 = lax.iota(jnp.int32, 16) + g * 16
                        cols16 = sidx_v[pl.ds(g * 16, 16)] + i
                        vals = plsc.load_gather(rows_v, [rows16, cols16])
                        plsc.store_scatter(
                            out_v, [rows16, jnp.full((16,), i, jnp.int32)],
                            vals)
                    return 0

                lax.fori_loop(0, _D, body, 0)
                pltpu.sync_copy(out_v, out_h.at[pl.ds(base, _BPW)])

    return k(users, pos_items, neg_items, heads, pos_tails, neg_tails,
             ue4, ie4, ke4)


def _l2n(x):
    n = jnp.sqrt(jnp.sum(x * x, axis=1, keepdims=True))
    return x / jnp.maximum(n, 1e-12)


def _tc_body(u_ref, pi_ref, pkg_ref, ni_ref, nkg_ref, h_ref, ptr_ref, ntr_ref,
             krel_ref, rel_ref, w2_ref,
             pc_ref, nc_ref, hn_ref, rn_ref, ptn_ref, ntn_ref, pred_ref):
    i = pl.program_id(0)
    sl = pl.ds(i * _BB, _BB)

    pos_comb_full = pi_ref[...] + pkg_ref[...]
    pc_ref[...] = pi_ref[sl, :] + pkg_ref[sl, :]
    nc_ref[...] = ni_ref[...] + nkg_ref[...]

    pred_ref[...] = lax.dot_general(
        u_ref[...], pos_comb_full,
        dimension_numbers=(((1,), (1,)), ((), ())),
        preferred_element_type=jnp.float32)

    rd = _R * _D
    rel = rel_ref[...]
    lane = lax.broadcasted_iota(jnp.int32, (_BB, rd), 1)
    oh_exp = (jnp.broadcast_to(rel, (_BB, rd)) ==
              (lane // _D)).astype(jnp.float32)

    iota_r = lax.broadcasted_iota(jnp.int32, (_BB, _R), 1)
    oh_r = (rel == iota_r).astype(jnp.float32)
    rn_ref[...] = _l2n(jnp.dot(oh_r, krel_ref[...],
                               preferred_element_type=jnp.float32))

    ic = lax.broadcasted_iota(jnp.int32, (rd, _D), 0)
    io = lax.broadcasted_iota(jnp.int32, (rd, _D), 1)
    sel = ((ic % _D) == io).astype(jnp.float32)

    w2 = w2_ref[...]
    for x_ref, out_ref in ((h_ref, hn_ref), (ptr_ref, ptn_ref),
                           (ntr_ref, ntn_ref)):
        y = jnp.dot(x_ref[...], w2,
                    preferred_element_type=jnp.float32)
        proj = jnp.dot(y * oh_exp, sel,
                       preferred_element_type=jnp.float32)
        out_ref[...] = _l2n(proj)


def _tc_dense(u_e, pi, pkg, ni, nkg, h_raw, pt_raw, nt_raw, krel, rel2d, w2):
    blk = pl.BlockSpec((_BB, _D), lambda i: (i, 0))
    full = pl.BlockSpec((_B, _D), lambda i: (0, 0))
    return pl.pallas_call(
        _tc_body,
        grid=(_G,),
        in_specs=[
            blk,
            full,
            full,
            blk, blk,
            blk, blk, blk,
            pl.BlockSpec((_R, _D), lambda i: (0, 0)),
            pl.BlockSpec((_BB, 1), lambda i: (i, 0)),
            pl.BlockSpec((_D, _R * _D), lambda i: (0, 0)),
        ],
        out_specs=[
            blk, blk, blk, blk, blk, blk,
            pl.BlockSpec((_BB, _B), lambda i: (i, 0)),
        ],
        out_shape=[
            jax.ShapeDtypeStruct((_B, _D), jnp.float32),
            jax.ShapeDtypeStruct((_B, _D), jnp.float32),
            jax.ShapeDtypeStruct((_B, _D), jnp.float32),
            jax.ShapeDtypeStruct((_B, _D), jnp.float32),
            jax.ShapeDtypeStruct((_B, _D), jnp.float32),
            jax.ShapeDtypeStruct((_B, _D), jnp.float32),
            jax.ShapeDtypeStruct((_B, _B), jnp.float32),
        ],
    )(u_e, pi, pkg, ni, nkg, h_raw, pt_raw, nt_raw, krel, rel2d, w2)


def kernel(users, pos_items, neg_items, heads, relations, pos_tails, neg_tails,
           user_embed, item_embed, kg_entity_embed, kg_relation_embed,
           trans_W):
    ue4 = user_embed.reshape(-1, 128)
    ie4 = item_embed.reshape(-1, 128)
    ke4 = kg_entity_embed.reshape(-1, 128)
    u_e, pi, pkg, ni, nkg, h_raw, pt_raw, nt_raw = _sc_gather(
        users, pos_items, neg_items, heads, pos_tails, neg_tails,
        ue4, ie4, ke4)
    w2 = jnp.transpose(trans_W, (1, 0, 2)).reshape(_D, _R * _D)
    rel2d = relations.reshape(_B, 1)
    pos_comb, neg_comb, h_n, r_n, pt_n, nt_n, preds = _tc_dense(
        u_e, pi, pkg, ni, nkg, h_raw, pt_raw, nt_raw, kg_relation_embed,
        rel2d, w2)
    return (u_e, pos_comb, neg_comb, h_n, r_n, pt_n, nt_n, preds)


# speedup vs baseline: 4.4195x; 4.4195x over previous
import functools

import jax
import jax.numpy as jnp
from jax import lax
from jax.experimental import pallas as pl
from jax.experimental.pallas import tpu as pltpu
from jax.experimental.pallas import tpu_sc as plsc

_B = 4096
_D = 32
_R = 64
_RD = _R * _D
_NC = 2
_NS = 16
_NW = _NC * _NS
_BPW = _B // _NW

_BB = 512
_G = _B // _BB


_N = 1000000
_CH = 16


def _sc_gather(users, pos_items, neg_items, heads, pos_tails, neg_tails,
               ueT, ieT, keT):
    mesh = plsc.VectorSubcoreMesh(core_axis_name="c", subcore_axis_name="s")
    out_t = [jax.ShapeDtypeStruct((_D, _B), jnp.float32)] * 8

    @functools.partial(
        pl.kernel,
        mesh=mesh,
        out_type=out_t,
        compiler_params=pltpu.CompilerParams(needs_layout_passes=False),
        scratch_types=[
            pltpu.VMEM((_BPW,), jnp.int32),
            pltpu.VMEM((_CH, _D, 128), jnp.float32),
            pltpu.VMEM((_D, _BPW), jnp.float32),
            pltpu.SemaphoreType.DMA,
        ],
    )
    def k(users_h, pos_h, neg_h, heads_h, pt_h, nt_h,
          ue_h, ie_h, ke_h,
          u_o, pi_o, pkg_o, ni_o, nkg_o, h_o, pt_o, nt_o,
          idx_v, pan_v, fbuf, sem):
        wid = lax.axis_index("s") * _NC + lax.axis_index("c")
        base = wid * _BPW
        jobs = (
            (users_h, ((ue_h, u_o),)),
            (pos_h, ((ie_h, pi_o), (ke_h, pkg_o))),
            (neg_h, ((ie_h, ni_o), (ke_h, nkg_o))),
            (heads_h, ((ke_h, h_o),)),
            (pt_h, ((ke_h, pt_o),)),
            (nt_h, ((ke_h, nt_o),)),
        )
        f16a = lax.iota(jnp.int32, 16)
        f16b = f16a + 16
        for idx_h, tabouts in jobs:
            pltpu.sync_copy(idx_h.at[pl.ds(base, _BPW)], idx_v)
            for tab_h, out_h in tabouts:

                def chunk(g, _, tab_h=tab_h):
                    v16 = idx_v[pl.ds(g * _CH, _CH)]
                    rs = [v16[kk] for kk in range(_CH)]
                    pos = [pl.multiple_of((r >> 7) << 7, 128) for r in rs]
                    descs = [
                        pltpu.async_copy(
                            tab_h.at[:, pl.ds(pos[kk], 128)],
                            pan_v.at[kk], sem)
                        for kk in range(_CH)
                    ]
                    for d in descs:
                        d.wait()
                    for kk in range(_CH):
                        c = jnp.full((16,), rs[kk] & 127, jnp.int32)
                        j = jnp.full((16,), g * _CH + kk, jnp.int32)
                        va = plsc.load_gather(pan_v.at[kk], [f16a, c])
                        vb = plsc.load_gather(pan_v.at[kk], [f16b, c])
                        plsc.store_scatter(fbuf, [f16a, j], va)
                        plsc.store_scatter(fbuf, [f16b, j], vb)
                    return 0

                lax.fori_loop(0, _BPW // _CH, chunk, 0)
                pltpu.sync_copy(fbuf, out_h.at[:, pl.ds(base, _BPW)])

    return k(users, pos_items, neg_items, heads, pos_tails, neg_tails,
             ueT, ieT, keT)


def _l2nT(x):
    n = jnp.sqrt(jnp.sum(x * x, axis=0, keepdims=True))
    return x / jnp.maximum(n, 1e-12)


def _tc_body(u_ref, pif_ref, pkgf_ref, pi_ref, pkg_ref, ni_ref, nkg_ref,
             h_ref, ptr_ref, ntr_ref, krel_ref, rel_ref, w2_ref,
             pc_ref, nc_ref, hn_ref, rn_ref, ptn_ref, ntn_ref, pred_ref):
    comb_fullT = pif_ref[...] + pkgf_ref[...]
    pc_ref[...] = pi_ref[...] + pkg_ref[...]
    nc_ref[...] = ni_ref[...] + nkg_ref[...]

    uT = u_ref[...]
    pred_ref[...] = lax.dot_general(
        uT, comb_fullT,
        dimension_numbers=(((0,), (0,)), ((), ())),
        preferred_element_type=jnp.float32)

    relrow = rel_ref[0:1, :]
    subl = lax.broadcasted_iota(jnp.int32, (_RD, _BB), 0)
    oh_expT = (jnp.broadcast_to(relrow, (_RD, _BB)) ==
               (subl // _D)).astype(jnp.float32)

    iota_r = lax.broadcasted_iota(jnp.int32, (_R, _BB), 0)
    oh_rT = (jnp.broadcast_to(relrow, (_R, _BB)) ==
             iota_r).astype(jnp.float32)
    rn_ref[...] = _l2nT(jnp.dot(krel_ref[...], oh_rT,
                                preferred_element_type=jnp.float32))

    ic = lax.broadcasted_iota(jnp.int32, (_D, _RD), 1)
    io = lax.broadcasted_iota(jnp.int32, (_D, _RD), 0)
    selT = ((ic % _D) == io).astype(jnp.float32)

    w2T = w2_ref[...]
    for xT_ref, outT_ref in ((h_ref, hn_ref), (ptr_ref, ptn_ref),
                             (ntr_ref, ntn_ref)):
        yT = jnp.dot(w2T, xT_ref[...],
                     preferred_element_type=jnp.float32)
        projT = jnp.dot(selT, yT * oh_expT,
                        preferred_element_type=jnp.float32)
        outT_ref[...] = _l2nT(projT)


def _tc_dense(uT, piT, pkgT, niT, nkgT, hT, ptT, ntT, krelT, rel8, w2T):
    blk = pl.BlockSpec((_D, _BB), lambda i: (0, i))
    full = pl.BlockSpec((_D, _B), lambda i: (0, 0))
    outs = pl.pallas_call(
        _tc_body,
        grid=(_G,),
        in_specs=[
            blk,
            full, full,
            blk, blk,
            blk, blk,
            blk, blk, blk,
            pl.BlockSpec((_D, _R), lambda i: (0, 0)),
            pl.BlockSpec((8, _BB), lambda i: (0, i)),
            pl.BlockSpec((_RD, _D), lambda i: (0, 0)),
        ],
        out_specs=[
            blk, blk, blk, blk, blk, blk,
            pl.BlockSpec((_BB, _B), lambda i: (i, 0)),
        ],
        out_shape=[
            jax.ShapeDtypeStruct((_D, _B), jnp.float32),
            jax.ShapeDtypeStruct((_D, _B), jnp.float32),
            jax.ShapeDtypeStruct((_D, _B), jnp.float32),
            jax.ShapeDtypeStruct((_D, _B), jnp.float32),
            jax.ShapeDtypeStruct((_D, _B), jnp.float32),
            jax.ShapeDtypeStruct((_D, _B), jnp.float32),
            jax.ShapeDtypeStruct((_B, _B), jnp.float32),
        ],
    )(uT, piT, pkgT, piT, pkgT, niT, nkgT, hT, ptT, ntT, krelT, rel8, w2T)
    return outs


def kernel(users, pos_items, neg_items, heads, relations, pos_tails, neg_tails,
           user_embed, item_embed, kg_entity_embed, kg_relation_embed,
           trans_W):
    ueT = user_embed.T
    ieT = item_embed.T
    keT = kg_entity_embed.T
    uT, piT, pkgT, niT, nkgT, hT, ptT, ntT = _sc_gather(
        users, pos_items, neg_items, heads, pos_tails, neg_tails,
        ueT, ieT, keT)
    w2T = jnp.transpose(trans_W, (0, 2, 1)).reshape(_RD, _D)
    krelT = kg_relation_embed.T
    rel8 = jnp.broadcast_to(relations.reshape(1, _B), (8, _B))
    pcT, ncT, hnT, rnT, ptnT, ntnT, preds = _tc_dense(
        uT, piT, pkgT, niT, nkgT, hT, ptT, ntT, krelT, rel8, w2T)
    return (uT.T, pcT.T, ncT.T, hnT.T, rnT.T, ptnT.T, ntnT.T, preds)


# speedup vs baseline: 4.5607x; 1.0320x over previous
import functools

import jax
import jax.numpy as jnp
from jax import lax
from jax.experimental import pallas as pl
from jax.experimental.pallas import tpu as pltpu
from jax.experimental.pallas import tpu_sc as plsc

_B = 4096
_D = 32
_R = 64
_RD = _R * _D
_NC = 2
_NS = 16
_NW = _NC * _NS
_BPW = _B // _NW

_BB = 512
_G = _B // _BB


_N = 1000000
_CH = 16


def _sc_gather(users, pos_items, neg_items, heads, pos_tails, neg_tails,
               ueT, ieT, keT):
    mesh = plsc.VectorSubcoreMesh(core_axis_name="c", subcore_axis_name="s")
    out_t = [jax.ShapeDtypeStruct((_D, _B), jnp.float32)] * 8
    ngr = _BPW // _CH

    @functools.partial(
        pl.kernel,
        mesh=mesh,
        out_type=out_t,
        compiler_params=pltpu.CompilerParams(needs_layout_passes=False),
        scratch_types=[
            pltpu.VMEM((_BPW,), jnp.int32),
            pltpu.VMEM((2, 8, _D, 128), jnp.float32),
            pltpu.VMEM((_D, _BPW), jnp.float32),
            pltpu.SemaphoreType.DMA,
            pltpu.SemaphoreType.DMA,
        ],
    )
    def k(users_h, pos_h, neg_h, heads_h, pt_h, nt_h,
          ue_h, ie_h, ke_h,
          u_o, pi_o, pkg_o, ni_o, nkg_o, h_o, pt_o, nt_o,
          idx_v, pan_v, fbuf, sem0, sem1):
        wid = lax.axis_index("s") * _NC + lax.axis_index("c")
        base = wid * _BPW
        sems = (sem0, sem1)
        jobs = (
            (users_h, ((ue_h, u_o),)),
            (pos_h, ((ie_h, pi_o), (ke_h, pkg_o))),
            (neg_h, ((ie_h, ni_o), (ke_h, nkg_o))),
            (heads_h, ((ke_h, h_o),)),
            (pt_h, ((ke_h, pt_o),)),
            (nt_h, ((ke_h, nt_o),)),
        )
        f16a = lax.iota(jnp.int32, 16)
        f16b = f16a + 16

        for idx_h, tabouts in jobs:
            pltpu.sync_copy(idx_h.at[pl.ds(base, _BPW)], idx_v)
            for tab_h, out_h in tabouts:

                def fire(rs8, slot, tab_h=tab_h):
                    for kk in range(8):
                        po = pl.multiple_of((rs8[kk] >> 7) << 7, 128)
                        pltpu.async_copy(tab_h.at[:, pl.ds(po, 128)],
                                         pan_v.at[slot, kk], sems[slot])

                def drain_extract(rs8, slot, jbase, tab_h=tab_h):
                    for kk in range(8):
                        pltpu.make_async_copy(tab_h.at[:, pl.ds(0, 128)],
                                              pan_v.at[slot, kk],
                                              sems[slot]).wait()
                    for kk in range(8):
                        c = jnp.full((16,), rs8[kk] & 127, jnp.int32)
                        j = jnp.full((16,), jbase + kk, jnp.int32)
                        va = plsc.load_gather(pan_v.at[slot, kk], [f16a, c])
                        vb = plsc.load_gather(pan_v.at[slot, kk], [f16b, c])
                        plsc.store_scatter(fbuf, [f16a, j], va)
                        plsc.store_scatter(fbuf, [f16b, j], vb)

                v0 = idx_v[pl.ds(0, _CH)]
                fire([v0[kk] for kk in range(8)], 0)

                def chunk(g, _):
                    v16 = idx_v[pl.ds(g * _CH, _CH)]
                    rs = [v16[kk] for kk in range(_CH)]
                    fire(rs[8:], 1)
                    drain_extract(rs[:8], 0, g * _CH)

                    @pl.when(g < ngr - 1)
                    def _():
                        vn = idx_v[pl.ds((g + 1) * _CH, _CH)]
                        fire([vn[kk] for kk in range(8)], 0)

                    drain_extract(rs[8:], 1, g * _CH + 8)
                    return 0

                lax.fori_loop(0, ngr, chunk, 0)
                pltpu.sync_copy(fbuf, out_h.at[:, pl.ds(base, _BPW)])

    return k(users, pos_items, neg_items, heads, pos_tails, neg_tails,
             ueT, ieT, keT)


def _l2nT(x):
    n = jnp.sqrt(jnp.sum(x * x, axis=0, keepdims=True))
    return x / jnp.maximum(n, 1e-12)


def _tc_body(u_ref, pif_ref, pkgf_ref, pi_ref, pkg_ref, ni_ref, nkg_ref,
             h_ref, ptr_ref, ntr_ref, krel_ref, rel_ref, w2_ref,
             pc_ref, nc_ref, hn_ref, rn_ref, ptn_ref, ntn_ref, pred_ref):
    comb_fullT = pif_ref[...] + pkgf_ref[...]
    pc_ref[...] = pi_ref[...] + pkg_ref[...]
    nc_ref[...] = ni_ref[...] + nkg_ref[...]

    uT = u_ref[...]
    pred_ref[...] = lax.dot_general(
        uT, comb_fullT,
        dimension_numbers=(((0,), (0,)), ((), ())),
        preferred_element_type=jnp.float32)

    relrow = rel_ref[0:1, :]
    subl = lax.broadcasted_iota(jnp.int32, (_RD, _BB), 0)
    oh_expT = (jnp.broadcast_to(relrow, (_RD, _BB)) ==
               (subl // _D)).astype(jnp.float32)

    iota_r = lax.broadcasted_iota(jnp.int32, (_R, _BB), 0)
    oh_rT = (jnp.broadcast_to(relrow, (_R, _BB)) ==
             iota_r).astype(jnp.float32)
    rn_ref[...] = _l2nT(jnp.dot(krel_ref[...], oh_rT,
                                preferred_element_type=jnp.float32))

    ic = lax.broadcasted_iota(jnp.int32, (_D, _RD), 1)
    io = lax.broadcasted_iota(jnp.int32, (_D, _RD), 0)
    selT = ((ic % _D) == io).astype(jnp.float32)

    w2T = w2_ref[...]
    for xT_ref, outT_ref in ((h_ref, hn_ref), (ptr_ref, ptn_ref),
                             (ntr_ref, ntn_ref)):
        yT = jnp.dot(w2T, xT_ref[...],
                     preferred_element_type=jnp.float32)
        projT = jnp.dot(selT, yT * oh_expT,
                        preferred_element_type=jnp.float32)
        outT_ref[...] = _l2nT(projT)


def _tc_dense(uT, piT, pkgT, niT, nkgT, hT, ptT, ntT, krelT, rel8, w2T):
    blk = pl.BlockSpec((_D, _BB), lambda i: (0, i))
    full = pl.BlockSpec((_D, _B), lambda i: (0, 0))
    outs = pl.pallas_call(
        _tc_body,
        grid=(_G,),
        in_specs=[
            blk,
            full, full,
            blk, blk,
            blk, blk,
            blk, blk, blk,
            pl.BlockSpec((_D, _R), lambda i: (0, 0)),
            pl.BlockSpec((8, _BB), lambda i: (0, i)),
            pl.BlockSpec((_RD, _D), lambda i: (0, 0)),
        ],
        out_specs=[
            blk, blk, blk, blk, blk, blk,
            pl.BlockSpec((_BB, _B), lambda i: (i, 0)),
        ],
        out_shape=[
            jax.ShapeDtypeStruct((_D, _B), jnp.float32),
            jax.ShapeDtypeStruct((_D, _B), jnp.float32),
            jax.ShapeDtypeStruct((_D, _B), jnp.float32),
            jax.ShapeDtypeStruct((_D, _B), jnp.float32),
            jax.ShapeDtypeStruct((_D, _B), jnp.float32),
            jax.ShapeDtypeStruct((_D, _B), jnp.float32),
            jax.ShapeDtypeStruct((_B, _B), jnp.float32),
        ],
    )(uT, piT, pkgT, piT, pkgT, niT, nkgT, hT, ptT, ntT, krelT, rel8, w2T)
    return outs


def kernel(users, pos_items, neg_items, heads, relations, pos_tails, neg_tails,
           user_embed, item_embed, kg_entity_embed, kg_relation_embed,
           trans_W):
    ueT = user_embed.T
    ieT = item_embed.T
    keT = kg_entity_embed.T
    uT, piT, pkgT, niT, nkgT, hT, ptT, ntT = _sc_gather(
        users, pos_items, neg_items, heads, pos_tails, neg_tails,
        ueT, ieT, keT)
    w2T = jnp.transpose(trans_W, (0, 2, 1)).reshape(_RD, _D)
    krelT = kg_relation_embed.T
    rel8 = jnp.broadcast_to(relations.reshape(1, _B), (8, _B))
    pcT, ncT, hnT, rnT, ptnT, ntnT, preds = _tc_dense(
        uT, piT, pkgT, niT, nkgT, hT, ptT, ntT, krelT, rel8, w2T)
    return (uT.T, pcT.T, ncT.T, hnT.T, rnT.T, ptnT.T, ntnT.T, preds)
